# baseline (speedup 1.0000x reference)
import jax
import jax.numpy as jnp
from jax import lax
from jax.experimental import pallas as pl

NN = 512
E_TOT = 16384
NEG = 0.2
HEADS = 4
DH = 32
CH = 2048

_HI = lax.Precision.HIGHEST


def _dot16(a, b, dims=None):
    if dims is None:
        dims = (((a.ndim - 1,), (0,)), ((), ()))
    return lax.dot_general(a.astype(jnp.bfloat16), b.astype(jnp.bfloat16),
                           dims, preferred_element_type=jnp.float32)


def _mlp_f(x, p):
    w0, b0, w1, b1, g, be, a1, w2, b2, w3, b3, a2 = p
    x1 = _dot16(x, w0) + b0
    h = _dot16(x1, w1) + b1
    m = jnp.mean(h, axis=-1, keepdims=True)
    v = jnp.mean((h - m) * (h - m), axis=-1, keepdims=True)
    h = (h - m) / jnp.sqrt(v + 1e-5) * g + be
    h = jnp.where(h > 0, h, a1[0, 0] * h)
    h = _dot16(h, w2) + b2
    x2 = h + x1
    x3 = _dot16(x2, w3) + b3
    return jnp.where(x3 > 0, x3, a2[0, 0] * x3)


def _gat_heads(x, W, al, ar, b, C):
    h = _dot16(x, W)
    outs = []
    M = None
    if C is not None:
        M = C > 0.0
    for hh in range(HEADS):
        hch = h[:, hh * DH:(hh + 1) * DH]
        al_h = al[:, hh * DH:(hh + 1) * DH]
        ar_h = ar[:, hh * DH:(hh + 1) * DH]
        el = lax.dot_general(al_h, hch, (((1,), (1,)), ((), ())),
                             preferred_element_type=jnp.float32,
                             precision=_HI)
        er = lax.dot_general(hch, ar_h, (((1,), (1,)), ((), ())),
                             preferred_element_type=jnp.float32,
                             precision=_HI)
        e = er + el
        e = jnp.where(e > 0, e, NEG * e)
        if C is None:
            emax = jnp.max(e, axis=1, keepdims=True)
            P = jnp.exp(e - emax)
            denom = jnp.sum(P, axis=1, keepdims=True)
            alpha = P / denom
            outs.append(_dot16(alpha, hch))
        else:
            emax = jnp.max(jnp.where(M, e, -jnp.inf), axis=1, keepdims=True)
            emax = jnp.where(emax > -1e38, emax, 0.0)
            P = C * jnp.exp(jnp.minimum(e - emax, 0.0))
            denom = jnp.sum(P, axis=1, keepdims=True)
            alpha = P / (denom + 1e-9)
            outs.append(jnp.dot(alpha, hch,
                                preferred_element_type=jnp.float32,
                                precision=_HI))
    return jnp.concatenate(outs, axis=1) + b


def _count_matrix(src_ref, dst_ref):
    iota = lax.broadcasted_iota(jnp.int32, (CH, NN), 1)

    def body(ci, C):
        s = src_ref[pl.ds(ci * CH, CH), :]
        d = dst_ref[pl.ds(ci * CH, CH), :]
        S = (s == iota).astype(jnp.bfloat16)
        D = (d == iota).astype(jnp.bfloat16)
        return C + lax.dot_general(D, S, (((0,), (0,)), ((), ())),
                                   preferred_element_type=jnp.float32)

    return lax.fori_loop(0, E_TOT // CH, body,
                         jnp.zeros((NN, NN), jnp.float32))


def _gcn_f(x, W, b, C):
    ones = jnp.ones((NN, 1), jnp.float32)
    deg_out = lax.dot_general(C, ones, (((0,), (0,)), ((), ())),
                              preferred_element_type=jnp.float32,
                              precision=_HI)
    deg_in = jnp.dot(C, ones, preferred_element_type=jnp.float32,
                     precision=_HI)
    feat = x * lax.rsqrt(jnp.maximum(deg_out, 1.0))
    agg = jnp.dot(C, feat, preferred_element_type=jnp.float32,
                  precision=_HI)
    rst = agg * lax.rsqrt(jnp.maximum(deg_in, 1.0))
    return _dot16(rst, W) + b


def _kernel_body(*refs):
    (fmri, dti, srcf, dstf, srcd, dstd,
     Wgf, bgf, Wgd, bgd) = refs[:10]
    it = iter(refs[10:])
    mlp_in = [next(it)[...] for _ in range(12)]
    mlp_mid = [next(it)[...] for _ in range(12)]
    mlp_of = [next(it)[...] for _ in range(12)]
    mlp_od = [next(it)[...] for _ in range(12)]
    gats = []
    for _ in range(4):
        gats.append([next(it)[...] for _ in range(4)])
    ff_o, fd_o, hf_o, hd_o, sh_o = [next(it) for _ in range(5)]

    Cf = _count_matrix(srcf, dstf)
    Cd = _count_matrix(srcd, dstd)

    fm = _gcn_f(fmri[...], Wgf[...], bgf[...], Cf)
    dt = _gcn_f(dti[...], Wgd[...], bgd[...], Cd)

    sh_o[...] = (fm + dt) * 0.5

    S1 = _dot16(fm, dt, (((1,), (1,)), ((), ())))
    S2 = _dot16(dt, fm, (((1,), (1,)), ((), ())))
    S0 = (S1 + S2) * 0.5
    S0 = jnp.clip(S0, -1e10, 1e10)
    h_t = _mlp_f(S0, mlp_in)

    d0 = _gat_heads(h_t, *gats[0], None)
    d1 = _gat_heads(d0, *gats[1], None)
    h_t = _mlp_f(d1, mlp_mid)
    h_t = h_t + d1
    h_t = _gat_heads(h_t, *gats[2], None)
    h_t = h_t + d0
    h_f = _gat_heads(h_t, *gats[3], Cf)
    h_d = _gat_heads(h_t, *gats[3], Cd)
    hf_o[...] = h_f
    hd_o[...] = h_d
    ff_o[...] = _mlp_f(h_f, mlp_of)
    fd_o[...] = _mlp_f(h_d, mlp_od)


def _prep_mlp(p):
    w0, b0, w1, b1, g, be, a1, w2, b2, w3, b3, a2 = p
    r = lambda v: jnp.asarray(v, jnp.float32).reshape(1, -1)
    return [w0, r(b0), w1, r(b1), r(g), r(be), r(a1), w2, r(b2), w3, r(b3),
            r(a2)]


def _prep_gat(p):
    W, al, ar, b = p
    return [W, al.reshape(1, -1), ar.reshape(1, -1), b.reshape(1, -1)]


def kernel(fmri_data, dti_data, time_embed, edge_index_fmri, edge_index_dti,
           gcn_fmri, gcn_dti, mlp_in_t, mlp_middle, mlp_out_fmri,
           mlp_out_dti, down_gats, up_gats):
    del time_embed
    srcf = edge_index_fmri[0].reshape(E_TOT, 1)
    dstf = edge_index_fmri[1].reshape(E_TOT, 1)
    srcd = edge_index_dti[0].reshape(E_TOT, 1)
    dstd = edge_index_dti[1].reshape(E_TOT, 1)

    ops = [fmri_data, dti_data, srcf, dstf, srcd, dstd,
           gcn_fmri[0], gcn_fmri[1].reshape(1, -1),
           gcn_dti[0], gcn_dti[1].reshape(1, -1)]
    ops += _prep_mlp(mlp_in_t)
    ops += _prep_mlp(mlp_middle)
    ops += _prep_mlp(mlp_out_fmri)
    ops += _prep_mlp(mlp_out_dti)
    for g in (down_gats[0], down_gats[1], up_gats[0], up_gats[1]):
        ops += _prep_gat(g)

    out_shapes = (
        jax.ShapeDtypeStruct((NN, 128), jnp.float32),
        jax.ShapeDtypeStruct((NN, 128), jnp.float32),
        jax.ShapeDtypeStruct((NN, 128), jnp.float32),
        jax.ShapeDtypeStruct((NN, 128), jnp.float32),
        jax.ShapeDtypeStruct((NN, NN), jnp.float32),
    )

    ff, fd, hf, hd, sh = pl.pallas_call(
        _kernel_body,
        out_shape=out_shapes,
    )(*ops)

    out_hidden = jnp.concatenate(
        [hf.reshape(NN, HEADS, DH), hd.reshape(NN, HEADS, DH)], axis=-1)
    return ff, fd, out_hidden, sh


# speedup vs baseline: 1.2055x; 1.2055x over previous
import functools

import jax
import jax.numpy as jnp
from jax import lax
from jax.experimental import pallas as pl
from jax.experimental.pallas import tpu as pltpu
from jax.experimental.pallas import tpu_sc as plsc

NN = 512
E_TOT = 16384
NEG = 0.2
HEADS = 4
DH = 32

_HI = lax.Precision.HIGHEST


SC_NC = 2
SC_NS = 16
SC_NW = SC_NC * SC_NS
ROWS = NN // SC_NW
VCH = 16

_sc_mesh = plsc.VectorSubcoreMesh(core_axis_name="c", subcore_axis_name="s")


@functools.partial(
    pl.kernel, mesh=_sc_mesh,
    compiler_params=pltpu.CompilerParams(needs_layout_passes=False),
    out_type=(jax.ShapeDtypeStruct((NN * NN,), jnp.float32),
              jax.ShapeDtypeStruct((NN * NN,), jnp.float32)),
    scratch_types=[
        pltpu.VMEM((E_TOT,), jnp.int32),
        pltpu.VMEM((E_TOT,), jnp.int32),
        pltpu.VMEM((ROWS * NN,), jnp.float32),
    ],
)
def _count_sc(srcf_hbm, dstf_hbm, srcd_hbm, dstd_hbm, cf_hbm, cd_hbm,
              src_v, dst_v, blk):
    cid = lax.axis_index("c")
    sid = lax.axis_index("s")
    wid = sid * SC_NC + cid
    lo = wid * ROWS
    ones = jnp.ones((VCH,), jnp.float32)
    zeros = jnp.zeros((VCH,), jnp.float32)

    for (sh, dh, oh) in ((srcf_hbm, dstf_hbm, cf_hbm),
                         (srcd_hbm, dstd_hbm, cd_hbm)):
        pltpu.sync_copy(sh, src_v)
        pltpu.sync_copy(dh, dst_v)

        def zbody(j, _):
            blk[pl.ds(j * VCH, VCH)] = zeros
            return _
        lax.fori_loop(0, ROWS * NN // VCH, zbody, 0)

        def body(i, _):
            s = src_v[pl.ds(i * VCH, VCH)]
            d = dst_v[pl.ds(i * VCH, VCH)]
            msk = (d >= lo) & (d < lo + ROWS)
            idx = (d - lo) * NN + s
            plsc.addupdate_scatter(blk, [idx], ones, mask=msk)
            return _
        lax.fori_loop(0, E_TOT // VCH, body, 0)

        pltpu.sync_copy(blk, oh.at[pl.ds(lo * NN, ROWS * NN)])


def _dot16(a, b, dims=None):
    if dims is None:
        dims = (((a.ndim - 1,), (0,)), ((), ()))
    return lax.dot_general(a.astype(jnp.bfloat16), b.astype(jnp.bfloat16),
                           dims, preferred_element_type=jnp.float32)


def _mlp_f(x, p):
    w0, b0, w1, b1, g, be, a1, w2, b2, w3, b3, a2 = p
    x1 = _dot16(x, w0) + b0
    h = _dot16(x1, w1) + b1
    m = jnp.mean(h, axis=-1, keepdims=True)
    v = jnp.mean((h - m) * (h - m), axis=-1, keepdims=True)
    h = (h - m) / jnp.sqrt(v + 1e-5) * g + be
    h = jnp.where(h > 0, h, a1[0, 0] * h)
    h = _dot16(h, w2) + b2
    x2 = h + x1
    x3 = _dot16(x2, w3) + b3
    return jnp.where(x3 > 0, x3, a2[0, 0] * x3)


def _gat_heads(x, W, al, ar, b, C):
    h = _dot16(x, W)
    outs = []
    M = None
    if C is not None:
        M = C > 0.0
    for hh in range(HEADS):
        hch = h[:, hh * DH:(hh + 1) * DH]
        al_h = al[:, hh * DH:(hh + 1) * DH]
        ar_h = ar[:, hh * DH:(hh + 1) * DH]
        el = lax.dot_general(al_h, hch, (((1,), (1,)), ((), ())),
                             preferred_element_type=jnp.float32,
                             precision=_HI)
        er = lax.dot_general(hch, ar_h, (((1,), (1,)), ((), ())),
                             preferred_element_type=jnp.float32,
                             precision=_HI)
        e = er + el
        e = jnp.where(e > 0, e, NEG * e)
        if C is None:
            emax = jnp.max(e, axis=1, keepdims=True)
            P = jnp.exp(e - emax)
            denom = jnp.sum(P, axis=1, keepdims=True)
            alpha = P / denom
            outs.append(_dot16(alpha, hch))
        else:
            emax = jnp.max(jnp.where(M, e, -jnp.inf), axis=1, keepdims=True)
            emax = jnp.where(emax > -1e38, emax, 0.0)
            P = C * jnp.exp(jnp.minimum(e - emax, 0.0))
            denom = jnp.sum(P, axis=1, keepdims=True)
            alpha = P / (denom + 1e-9)
            outs.append(jnp.dot(alpha, hch,
                                preferred_element_type=jnp.float32,
                                precision=_HI))
    return jnp.concatenate(outs, axis=1) + b


def _gcn_f(x, W, b, C):
    ones = jnp.ones((NN, 1), jnp.float32)
    deg_out = lax.dot_general(C, ones, (((0,), (0,)), ((), ())),
                              preferred_element_type=jnp.float32,
                              precision=_HI)
    deg_in = jnp.dot(C, ones, preferred_element_type=jnp.float32,
                     precision=_HI)
    feat = x * lax.rsqrt(jnp.maximum(deg_out, 1.0))
    agg = jnp.dot(C, feat, preferred_element_type=jnp.float32,
                  precision=_HI)
    rst = agg * lax.rsqrt(jnp.maximum(deg_in, 1.0))
    return _dot16(rst, W) + b


def _kernel_body(*refs):
    (fmri, dti, cf_r, cd_r, Wgf, bgf, Wgd, bgd) = refs[:8]
    it = iter(refs[8:])
    mlp_in = [next(it)[...] for _ in range(12)]
    mlp_mid = [next(it)[...] for _ in range(12)]
    mlp_of = [next(it)[...] for _ in range(12)]
    mlp_od = [next(it)[...] for _ in range(12)]
    gats = []
    for _ in range(4):
        gats.append([next(it)[...] for _ in range(4)])
    ff_o, fd_o, hf_o, hd_o, sh_o = [next(it) for _ in range(5)]

    Cf = cf_r[...]
    Cd = cd_r[...]

    fm = _gcn_f(fmri[...], Wgf[...], bgf[...], Cf)
    dt = _gcn_f(dti[...], Wgd[...], bgd[...], Cd)

    sh_o[...] = (fm + dt) * 0.5

    S1 = _dot16(fm, dt, (((1,), (1,)), ((), ())))
    S2 = _dot16(dt, fm, (((1,), (1,)), ((), ())))
    S0 = (S1 + S2) * 0.5
    S0 = jnp.clip(S0, -1e10, 1e10)
    h_t = _mlp_f(S0, mlp_in)

    d0 = _gat_heads(h_t, *gats[0], None)
    d1 = _gat_heads(d0, *gats[1], None)
    h_t = _mlp_f(d1, mlp_mid)
    h_t = h_t + d1
    h_t = _gat_heads(h_t, *gats[2], None)
    h_t = h_t + d0
    h_f = _gat_heads(h_t, *gats[3], Cf)
    h_d = _gat_heads(h_t, *gats[3], Cd)
    hf_o[...] = h_f
    hd_o[...] = h_d
    ff_o[...] = _mlp_f(h_f, mlp_of)
    fd_o[...] = _mlp_f(h_d, mlp_od)


def _prep_mlp(p):
    w0, b0, w1, b1, g, be, a1, w2, b2, w3, b3, a2 = p
    r = lambda v: jnp.asarray(v, jnp.float32).reshape(1, -1)
    return [w0, r(b0), w1, r(b1), r(g), r(be), r(a1), w2, r(b2), w3, r(b3),
            r(a2)]


def _prep_gat(p):
    W, al, ar, b = p
    return [W, al.reshape(1, -1), ar.reshape(1, -1), b.reshape(1, -1)]


def kernel(fmri_data, dti_data, time_embed, edge_index_fmri, edge_index_dti,
           gcn_fmri, gcn_dti, mlp_in_t, mlp_middle, mlp_out_fmri,
           mlp_out_dti, down_gats, up_gats):
    del time_embed

    cf_flat, cd_flat = _count_sc(
        edge_index_fmri[0], edge_index_fmri[1],
        edge_index_dti[0], edge_index_dti[1])
    Cf = cf_flat.reshape(NN, NN)
    Cd = cd_flat.reshape(NN, NN)

    ops = [fmri_data, dti_data, Cf, Cd,
           gcn_fmri[0], gcn_fmri[1].reshape(1, -1),
           gcn_dti[0], gcn_dti[1].reshape(1, -1)]
    ops += _prep_mlp(mlp_in_t)
    ops += _prep_mlp(mlp_middle)
    ops += _prep_mlp(mlp_out_fmri)
    ops += _prep_mlp(mlp_out_dti)
    for g in (down_gats[0], down_gats[1], up_gats[0], up_gats[1]):
        ops += _prep_gat(g)

    out_shapes = (
        jax.ShapeDtypeStruct((NN, 128), jnp.float32),
        jax.ShapeDtypeStruct((NN, 128), jnp.float32),
        jax.ShapeDtypeStruct((NN, 128), jnp.float32),
        jax.ShapeDtypeStruct((NN, 128), jnp.float32),
        jax.ShapeDtypeStruct((NN, NN), jnp.float32),
    )

    ff, fd, hf, hd, sh = pl.pallas_call(
        _kernel_body,
        out_shape=out_shapes,
    )(*ops)

    out_hidden = jnp.concatenate(
        [hf.reshape(NN, HEADS, DH), hd.reshape(NN, HEADS, DH)], axis=-1)
    return ff, fd, out_hidden, sh


# speedup vs baseline: 1.2200x; 1.0120x over previous
import functools

import jax
import jax.numpy as jnp
from jax import lax
from jax.experimental import pallas as pl
from jax.experimental.pallas import tpu as pltpu
from jax.experimental.pallas import tpu_sc as plsc

NN = 512
E_TOT = 16384
NEG = 0.2
HEADS = 4
DH = 32

_HI = lax.Precision.HIGHEST


SC_NC = 2
SC_NS = 16
SC_NW = SC_NC * SC_NS
ROWS = NN // SC_NW
VCH = 16

_sc_mesh = plsc.VectorSubcoreMesh(core_axis_name="c", subcore_axis_name="s")


@functools.partial(
    pl.kernel, mesh=_sc_mesh,
    compiler_params=pltpu.CompilerParams(needs_layout_passes=False),
    out_type=(jax.ShapeDtypeStruct((NN * NN,), jnp.float32),
              jax.ShapeDtypeStruct((NN * NN,), jnp.float32)),
    scratch_types=[
        pltpu.VMEM((E_TOT,), jnp.int32),
        pltpu.VMEM((E_TOT,), jnp.int32),
        pltpu.VMEM((ROWS * NN,), jnp.float32),
    ],
)
def _count_sc(srcf_hbm, dstf_hbm, srcd_hbm, dstd_hbm, cf_hbm, cd_hbm,
              src_v, dst_v, blk):
    cid = lax.axis_index("c")
    sid = lax.axis_index("s")
    wid = sid * SC_NC + cid
    lo = wid * ROWS
    ones = jnp.ones((VCH,), jnp.float32)
    zeros = jnp.zeros((VCH,), jnp.float32)

    for (sh, dh, oh) in ((srcf_hbm, dstf_hbm, cf_hbm),
                         (srcd_hbm, dstd_hbm, cd_hbm)):
        pltpu.sync_copy(sh, src_v)
        pltpu.sync_copy(dh, dst_v)

        def zbody(j, _):
            blk[pl.ds(j * VCH, VCH)] = zeros
            return _
        lax.fori_loop(0, ROWS * NN // VCH, zbody, 0)

        def body(i, _):
            s = src_v[pl.ds(i * VCH, VCH)]
            d = dst_v[pl.ds(i * VCH, VCH)]
            msk = (d >= lo) & (d < lo + ROWS)
            idx = (d - lo) * NN + s
            plsc.addupdate_scatter(blk, [idx], ones, mask=msk)
            return _
        lax.fori_loop(0, E_TOT // VCH, body, 0)

        pltpu.sync_copy(blk, oh.at[pl.ds(lo * NN, ROWS * NN)])


def _dot16(a, b, dims=None):
    if dims is None:
        dims = (((a.ndim - 1,), (0,)), ((), ()))
    return lax.dot_general(a.astype(jnp.bfloat16), b.astype(jnp.bfloat16),
                           dims, preferred_element_type=jnp.float32)


def _mlp_f(x, p):
    w0, b0, w1, b1, g, be, a1, w2, b2, w3, b3, a2 = p
    x1 = _dot16(x, w0) + b0
    h = _dot16(x1, w1) + b1
    m = jnp.mean(h, axis=-1, keepdims=True)
    v = jnp.mean((h - m) * (h - m), axis=-1, keepdims=True)
    h = (h - m) / jnp.sqrt(v + 1e-5) * g + be
    h = jnp.where(h > 0, h, a1[0, 0] * h)
    h = _dot16(h, w2) + b2
    x2 = h + x1
    x3 = _dot16(x2, w3) + b3
    return jnp.where(x3 > 0, x3, a2[0, 0] * x3)


def _gat_heads(x, W, al, ar, b, C):
    h = _dot16(x, W)
    outs = []
    Mneg = None
    if C is not None:
        Mneg = jnp.where(C > 0.0, 0.0, -jnp.inf)
    for hh in range(HEADS):
        hch = h[:, hh * DH:(hh + 1) * DH]
        al_h = al[:, hh * DH:(hh + 1) * DH]
        ar_h = ar[:, hh * DH:(hh + 1) * DH]
        el = lax.dot_general(al_h, hch, (((1,), (1,)), ((), ())),
                             preferred_element_type=jnp.float32,
                             precision=_HI)
        er = lax.dot_general(hch, ar_h, (((1,), (1,)), ((), ())),
                             preferred_element_type=jnp.float32,
                             precision=_HI)
        e = er + el
        e = jnp.where(e > 0, e, NEG * e)
        if C is None:
            emax = jnp.max(e, axis=1, keepdims=True)
            P = jnp.exp(e - emax)
            denom = jnp.sum(P, axis=1, keepdims=True)
            outs.append(_dot16(P, hch) / denom)
        else:
            emax = jnp.max(e + Mneg, axis=1, keepdims=True)
            emax = jnp.where(emax > -1e38, emax, 0.0)
            P = C * jnp.exp(jnp.minimum(e - emax, 0.0))
            denom = jnp.sum(P, axis=1, keepdims=True)
            outs.append(jnp.dot(P, hch,
                                preferred_element_type=jnp.float32,
                                precision=_HI) / (denom + 1e-9))
    return jnp.concatenate(outs, axis=1) + b


def _gcn_f(x, W, b, C):
    ones = jnp.ones((NN, 1), jnp.float32)
    deg_out = lax.dot_general(C, ones, (((0,), (0,)), ((), ())),
                              preferred_element_type=jnp.float32,
                              precision=_HI)
    deg_in = jnp.dot(C, ones, preferred_element_type=jnp.float32,
                     precision=_HI)
    feat = x * lax.rsqrt(jnp.maximum(deg_out, 1.0))
    agg = jnp.dot(C, feat, preferred_element_type=jnp.float32,
                  precision=_HI)
    rst = agg * lax.rsqrt(jnp.maximum(deg_in, 1.0))
    return _dot16(rst, W) + b


def _kernel_body(*refs):
    (fmri, dti, cf_r, cd_r, Wgf, bgf, Wgd, bgd) = refs[:8]
    it = iter(refs[8:])
    mlp_in = [next(it)[...] for _ in range(12)]
    mlp_mid = [next(it)[...] for _ in range(12)]
    mlp_of = [next(it)[...] for _ in range(12)]
    mlp_od = [next(it)[...] for _ in range(12)]
    gats = []
    for _ in range(4):
        gats.append([next(it)[...] for _ in range(4)])
    ff_o, fd_o, hf_o, hd_o, sh_o = [next(it) for _ in range(5)]

    Cf = cf_r[...]
    Cd = cd_r[...]

    fm = _gcn_f(fmri[...], Wgf[...], bgf[...], Cf)
    dt = _gcn_f(dti[...], Wgd[...], bgd[...], Cd)

    sh_o[...] = (fm + dt) * 0.5

    S1 = _dot16(fm, dt, (((1,), (1,)), ((), ())))
    S2 = _dot16(dt, fm, (((1,), (1,)), ((), ())))
    S0 = (S1 + S2) * 0.5
    S0 = jnp.clip(S0, -1e10, 1e10)
    h_t = _mlp_f(S0, mlp_in)

    d0 = _gat_heads(h_t, *gats[0], None)
    d1 = _gat_heads(d0, *gats[1], None)
    h_t = _mlp_f(d1, mlp_mid)
    h_t = h_t + d1
    h_t = _gat_heads(h_t, *gats[2], None)
    h_t = h_t + d0
    h_f = _gat_heads(h_t, *gats[3], Cf)
    h_d = _gat_heads(h_t, *gats[3], Cd)
    hf_o[...] = h_f
    hd_o[...] = h_d
    ff_o[...] = _mlp_f(h_f, mlp_of)
    fd_o[...] = _mlp_f(h_d, mlp_od)


def _prep_mlp(p):
    w0, b0, w1, b1, g, be, a1, w2, b2, w3, b3, a2 = p
    r = lambda v: jnp.asarray(v, jnp.float32).reshape(1, -1)
    return [w0, r(b0), w1, r(b1), r(g), r(be), r(a1), w2, r(b2), w3, r(b3),
            r(a2)]


def _prep_gat(p):
    W, al, ar, b = p
    return [W, al.reshape(1, -1), ar.reshape(1, -1), b.reshape(1, -1)]


def kernel(fmri_data, dti_data, time_embed, edge_index_fmri, edge_index_dti,
           gcn_fmri, gcn_dti, mlp_in_t, mlp_middle, mlp_out_fmri,
           mlp_out_dti, down_gats, up_gats):
    del time_embed

    cf_flat, cd_flat = _count_sc(
        edge_index_fmri[0], edge_index_fmri[1],
        edge_index_dti[0], edge_index_dti[1])
    Cf = cf_flat.reshape(NN, NN)
    Cd = cd_flat.reshape(NN, NN)

    ops = [fmri_data, dti_data, Cf, Cd,
           gcn_fmri[0], gcn_fmri[1].reshape(1, -1),
           gcn_dti[0], gcn_dti[1].reshape(1, -1)]
    ops += _prep_mlp(mlp_in_t)
    ops += _prep_mlp(mlp_middle)
    ops += _prep_mlp(mlp_out_fmri)
    ops += _prep_mlp(mlp_out_dti)
    for g in (down_gats[0], down_gats[1], up_gats[0], up_gats[1]):
        ops += _prep_gat(g)

    out_shapes = (
        jax.ShapeDtypeStruct((NN, 128), jnp.float32),
        jax.ShapeDtypeStruct((NN, 128), jnp.float32),
        jax.ShapeDtypeStruct((NN, 128), jnp.float32),
        jax.ShapeDtypeStruct((NN, 128), jnp.float32),
        jax.ShapeDtypeStruct((NN, NN), jnp.float32),
    )

    ff, fd, hf, hd, sh = pl.pallas_call(
        _kernel_body,
        out_shape=out_shapes,
    )(*ops)

    out_hidden = jnp.concatenate(
        [hf.reshape(NN, HEADS, DH), hd.reshape(NN, HEADS, DH)], axis=-1)
    return ff, fd, out_hidden, sh
